# Initial kernel scaffold; baseline (speedup 1.0000x reference)
import jax
import jax.numpy as jnp
from jax.experimental import pallas as pl


def kernel(protein_x, protein_edge, protein_batch, mol_x, mol_edge, mol_batch, pW1, pb1, pg1, pbe1, pW2, pb2, pg2, pbe2, pW3, pb3, mW1, mb1, mg1, mbe1, mW2, mb2, mg2, mbe2, mW3, mb3, fW1, fb1, fW2, fb2, fW3, fb3, fW4, fb4):
    raise NotImplementedError("write your pallas kernel here")


# speedup vs baseline: 6.6089x; 6.6089x over previous
import functools
import math

import jax
import jax.numpy as jnp
from jax import lax
from jax.experimental import pallas as pl
from jax.experimental.pallas import tpu as pltpu
from jax.experimental.pallas import tpu_sc as plsc

_NC, _NS = 2, 16
_K = 2048
_JW = 128
_NJ = _K // _JW
_SW = 16


def _sc_mesh():
    return plsc.VectorSubcoreMesh(
        core_axis_name="c", subcore_axis_name="s",
        num_cores=_NC, num_subcores=_NS)


def _zero_rows(buf, nrows, width):
    z = jnp.zeros((16,), jnp.float32)

    def body(i, _):
        for w0 in range(0, width, 16):
            buf[i, pl.ds(w0, 16)] = z
        return 0

    lax.fori_loop(0, nrows, body, 0)


def _zero_acc(acc, zbuf, row0, nrows, zrows):
    off = 0
    while off < nrows:
        n = min(zrows, nrows - off)
        pltpu.sync_copy(zbuf.at[pl.ds(0, n)], acc.at[pl.ds(row0 + off, n)])
        off += n


def _readout(acc, out, N, R, s):
    ntiles_full = N // R
    rem = N - ntiles_full * R

    @pl.when(s < ntiles_full)
    def _():
        pltpu.sync_copy(acc.at[pl.ds(s * R, R)], out.at[pl.ds(s * R, R)])

    if rem:
        @pl.when(s == ntiles_full)
        def _():
            pltpu.sync_copy(acc.at[pl.ds(ntiles_full * R, rem)],
                            out.at[pl.ds(ntiles_full * R, rem)])


@functools.lru_cache(maxsize=None)
def _make_sc_degree(N, EPAD):
    CCH = EPAD // (_NS * _K)
    R = (-(-(N + 1) // _NS) + 7) // 8 * 8
    ACC = R * _NS

    @functools.partial(
        pl.kernel,
        out_type=(jax.ShapeDtypeStruct((N, _SW), jnp.float32),
                  jax.ShapeDtypeStruct((N, _SW), jnp.float32)),
        mesh=_sc_mesh(),
        compiler_params=pltpu.CompilerParams(use_tc_tiling_on_sc=False),
        scratch_types=[
            pltpu.VMEM((_K, _SW), jnp.float32),
            pltpu.VMEM((_JW, _SW), jnp.float32),
            pltpu.VMEM((_NJ, _JW), jnp.int32),
            pltpu.VMEM_SHARED((ACC, _SW), jnp.float32),
        ],
    )
    def deg_kernel(dstA, dstB, outA, outB, zbuf, ones, idx_d, acc):
        c = lax.axis_index("c")
        s = lax.axis_index("s")

        _zero_rows(zbuf, _K, _SW)
        one = jnp.ones((16,), jnp.float32)

        def fill_ones(i, _):
            ones[i, pl.ds(0, 16)] = one
            return 0
        lax.fori_loop(0, _JW, fill_ones, 0)

        _zero_acc(acc, zbuf, s * R, R, _K)
        plsc.subcore_barrier()

        def run(dst2d, out):
            def chunk(i, _):
                rbase = (s * CCH + i) * _NJ
                pltpu.sync_copy(dst2d.at[pl.ds(rbase, _NJ)], idx_d)
                for j in range(_NJ):
                    pltpu.sync_copy(ones, acc.at[idx_d.at[j]], add=True)
                return 0
            lax.fori_loop(0, CCH, chunk, 0)
            plsc.subcore_barrier()
            _readout(acc, out, N, R, s)

        @pl.when(c == 0)
        def _():
            run(dstA, outA)

        @pl.when(c == 1)
        def _():
            run(dstB, outB)

    return deg_kernel


@functools.lru_cache(maxsize=None)
def _make_sc_scatter(N, EPAD, nslabs):
    CCH = EPAD // (_NS * _K)
    R = (-(-(N + 1) // _NS) + 7) // 8 * 8
    ACC = R * _NS
    out_type = tuple(jax.ShapeDtypeStruct((N, _SW), jnp.float32)
                     for _ in range(nslabs))

    @functools.partial(
        pl.kernel,
        out_type=out_type,
        mesh=_sc_mesh(),
        compiler_params=pltpu.CompilerParams(use_tc_tiling_on_sc=False),
        scratch_types=[
            pltpu.VMEM((_K, _SW), jnp.float32),
            pltpu.VMEM((_NJ, _JW), jnp.int32),
            pltpu.VMEM((_NJ, _JW), jnp.int32),
            pltpu.VMEM_SHARED((ACC, _SW), jnp.float32),
            pltpu.SemaphoreType.DMA,
        ],
    )
    def scatter_kernel(*args):
        tables = args[:nslabs]
        src2d = args[nslabs]
        dst2d = args[nslabs + 1]
        outs = args[nslabs + 2:2 * nslabs + 2]
        rows, idx_s, idx_d, acc, sem = args[2 * nslabs + 2:]

        c = lax.axis_index("c")
        s = lax.axis_index("s")

        def phase(table, out):
            _zero_rows(rows, _K, _SW)
            _zero_acc(acc, rows, s * R, R, _K)
            plsc.subcore_barrier()

            def chunk(i, _):
                rbase = (s * CCH + i) * _NJ
                pltpu.sync_copy(src2d.at[pl.ds(rbase, _NJ)], idx_s)
                pltpu.sync_copy(dst2d.at[pl.ds(rbase, _NJ)], idx_d)
                descs = []
                for j in range(_NJ):
                    descs.append(pltpu.async_copy(
                        table.at[idx_s.at[j]],
                        rows.at[pl.ds(j * _JW, _JW)], sem))
                for d in descs:
                    d.wait()
                for j in range(_NJ):
                    pltpu.sync_copy(rows.at[pl.ds(j * _JW, _JW)],
                                    acc.at[idx_d.at[j]], add=True)
                return 0

            lax.fori_loop(0, CCH, chunk, 0)
            plsc.subcore_barrier()
            _readout(acc, out, N, R, s)
            plsc.subcore_barrier()

        for p in range(nslabs // _NC):
            slab_lo = p * _NC
            @pl.when(c == 0)
            def _(slab=slab_lo):
                phase(tables[slab], outs[slab])

            @pl.when(c == 1)
            def _(slab=slab_lo + 1):
                phase(tables[slab], outs[slab])

    return scatter_kernel


_BM = 1000
_BN_S = 1.0 / math.sqrt(1.0 + 1e-5)


def _tc_layer1(x, W, cnt):
    N, Kp = x.shape
    F = W.shape[1]
    nsl = F // _SW
    grid = (N // _BM,)

    def body(x_ref, w_ref, cnt_ref, *outs):
        h_refs = outs[:nsl]
        dinv_ref = outs[nsl]
        deg = cnt_ref[...][:, 0:1] + 1.0
        dinv = lax.rsqrt(deg)
        u = jnp.dot(x_ref[...], w_ref[...],
                    preferred_element_type=jnp.float32)
        hp = u * dinv
        for si in range(nsl):
            h_refs[si][...] = hp[:, si * _SW:(si + 1) * _SW]
        dinv_ref[...] = dinv

    return pl.pallas_call(
        body,
        grid=grid,
        in_specs=[
            pl.BlockSpec((_BM, Kp), lambda i: (i, 0)),
            pl.BlockSpec((Kp, F), lambda i: (0, 0)),
            pl.BlockSpec((_BM, _SW), lambda i: (i, 0)),
        ],
        out_specs=([pl.BlockSpec((_BM, _SW), lambda i: (i, 0))
                    for _ in range(nsl)]
                   + [pl.BlockSpec((_BM, 1), lambda i: (i, 0))]),
        out_shape=([jax.ShapeDtypeStruct((N, _SW), jnp.float32)
                    for _ in range(nsl)]
                  + [jax.ShapeDtypeStruct((N, 1), jnp.float32)]),
    )(x, W, cnt)


def _tc_layer_mid(aggs, hs, dinv, b, g, be, W):
    nin = len(aggs)
    N = aggs[0].shape[0]
    F = W.shape[1]
    nout = F // _SW
    grid = (N // _BM,)

    def body(*refs):
        a = refs[0:nin]
        h = refs[nin:2 * nin]
        dv = refs[2 * nin]
        b_r = refs[2 * nin + 1]
        g_r = refs[2 * nin + 2]
        be_r = refs[2 * nin + 3]
        w_r = refs[2 * nin + 4]
        outs = refs[2 * nin + 5:]
        dinv_v = dv[...]
        bias = b_r[...]
        scale = g_r[...] * _BN_S
        shift = be_r[...]
        u = jnp.zeros((_BM, F), jnp.float32)
        for si in range(nin):
            lo, hi = si * _SW, (si + 1) * _SW
            z = dinv_v * (a[si][...] + h[si][...]) + bias[:, lo:hi]
            x = jnp.maximum(z * scale[:, lo:hi] + shift[:, lo:hi], 0.0)
            u = u + jnp.dot(x, w_r[...][lo:hi, :],
                            preferred_element_type=jnp.float32)
        hp = u * dinv_v
        for si in range(nout):
            outs[si][...] = hp[:, si * _SW:(si + 1) * _SW]

    return pl.pallas_call(
        body,
        grid=grid,
        in_specs=(
            [pl.BlockSpec((_BM, _SW), lambda i: (i, 0))
             for _ in range(2 * nin)]
            + [pl.BlockSpec((_BM, 1), lambda i: (i, 0)),
               pl.BlockSpec((1, nin * _SW), lambda i: (0, 0)),
               pl.BlockSpec((1, nin * _SW), lambda i: (0, 0)),
               pl.BlockSpec((1, nin * _SW), lambda i: (0, 0)),
               pl.BlockSpec((nin * _SW, F), lambda i: (0, 0))]
        ),
        out_specs=[pl.BlockSpec((_BM, _SW), lambda i: (i, 0))
                   for _ in range(nout)],
        out_shape=[jax.ShapeDtypeStruct((N, _SW), jnp.float32)
                   for _ in range(nout)],
    )(*aggs, *hs, dinv, b, g, be, W)


def _tc_pool(aggs, hs, dinv, b3, batch3d, G):
    N = aggs[0].shape[0]
    grid = (N // _BM,)
    nsl = len(aggs)

    def body(*refs):
        a = refs[0:nsl]
        h = refs[nsl:2 * nsl]
        dv = refs[2 * nsl]
        b_r = refs[2 * nsl + 1]
        bt = refs[2 * nsl + 2]
        pouts = refs[2 * nsl + 3:2 * nsl + 3 + nsl]
        cnt_o = refs[2 * nsl + 3 + nsl]

        @pl.when(pl.program_id(0) == 0)
        def _():
            for po in pouts:
                po[...] = jnp.zeros_like(po)
            cnt_o[...] = jnp.zeros_like(cnt_o)

        dinv_v = dv[...]
        bias = b_r[...]
        seg = bt[...].reshape(1, _BM)
        gid = lax.broadcasted_iota(jnp.int32, (G, _BM), 0)
        mask = (seg == gid).astype(jnp.float32)
        cnt_o[...] += jnp.sum(mask, axis=1, keepdims=True)
        for si in range(nsl):
            lo, hi = si * _SW, (si + 1) * _SW
            x4 = jnp.maximum(
                dinv_v * (a[si][...] + h[si][...]) + bias[:, lo:hi], 0.0)
            pouts[si][...] += jnp.dot(mask, x4,
                                      preferred_element_type=jnp.float32)

    return pl.pallas_call(
        body,
        grid=grid,
        in_specs=(
            [pl.BlockSpec((_BM, _SW), lambda i: (i, 0))
             for _ in range(2 * nsl)]
            + [pl.BlockSpec((_BM, 1), lambda i: (i, 0)),
               pl.BlockSpec((1, nsl * _SW), lambda i: (0, 0)),
               pl.BlockSpec((1, 1, _BM), lambda i: (i, 0, 0))]
        ),
        out_specs=([pl.BlockSpec((G, _SW), lambda i: (0, 0))
                    for _ in range(nsl)]
                   + [pl.BlockSpec((G, 1), lambda i: (0, 0))]),
        out_shape=([jax.ShapeDtypeStruct((G, _SW), jnp.float32)
                    for _ in range(nsl)]
                  + [jax.ShapeDtypeStruct((G, 1), jnp.float32)]),
    )(*aggs, *hs, dinv, b3, batch3d)


def _tc_head(poolP, cntP, poolM, cntM, fW1, fb1, fW2, fb2, fW3, fb3,
             fW4, fb4):
    G = cntP.shape[0]
    nsl = len(poolP)

    def body(*refs):
        pP = refs[0:nsl]
        cP = refs[nsl]
        pM = refs[nsl + 1:2 * nsl + 1]
        cM = refs[2 * nsl + 1]
        w1, b1, w2, b2, w3, b3, w4, b4, out = refs[2 * nsl + 2:]
        invP = 1.0 / jnp.maximum(cP[...], 1.0)
        invM = 1.0 / jnp.maximum(cM[...], 1.0)
        acc = jnp.zeros((G, 128), jnp.float32)
        half = nsl * _SW
        for si in range(nsl):
            acc += jnp.dot(pP[si][...] * invP,
                           w1[...][si * _SW:(si + 1) * _SW, :],
                           preferred_element_type=jnp.float32)
            acc += jnp.dot(pM[si][...] * invM,
                           w1[...][half + si * _SW:half + (si + 1) * _SW, :],
                           preferred_element_type=jnp.float32)
        h = jnp.maximum(acc + b1[...], 0.0)
        h = jnp.maximum(jnp.dot(h, w2[...], preferred_element_type=jnp.float32)
                        + b2[...], 0.0)
        h = jnp.maximum(jnp.dot(h, w3[...], preferred_element_type=jnp.float32)
                        + b3[...], 0.0)
        out[...] = (jnp.dot(h, w4[...], preferred_element_type=jnp.float32)
                    + b4[...])

    full = lambda shape: pl.BlockSpec(shape, lambda: tuple(0 for _ in shape))
    ins = ([full((G, _SW)) for _ in range(nsl)] + [full((G, 1))]
           + [full((G, _SW)) for _ in range(nsl)] + [full((G, 1))]
           + [full((256, 128)), full((1, 128)), full((128, 64)),
              full((1, 64)), full((64, 32)), full((1, 32)),
              full((32, 1)), full((1, 1))])
    return pl.pallas_call(
        body,
        grid=(),
        in_specs=ins,
        out_specs=full((G, 1)),
        out_shape=jax.ShapeDtypeStruct((G, 1), jnp.float32),
    )(*poolP, cntP, *poolM, cntM, fW1, fb1.reshape(1, -1), fW2,
      fb2.reshape(1, -1), fW3, fb3.reshape(1, -1), fW4, fb4.reshape(1, -1))


def _pad_edges(src, dst, N, EPAD):
    E = src.shape[0]
    pad = EPAD - E
    srcp = jnp.concatenate([src, jnp.zeros((pad,), jnp.int32)])
    dstp = jnp.concatenate([dst, jnp.full((pad,), N, jnp.int32)])
    return srcp.reshape(EPAD // _JW, _JW), dstp.reshape(EPAD // _JW, _JW)


def _branch(x, src2d, dst2d, cnt, W1, b1, g1, be1, W2, b2, g2, be2, W3, b3,
            batch3d, N, EPAD, G):
    sc4 = _make_sc_scatter(N, EPAD, 4)
    sc8 = _make_sc_scatter(N, EPAD, 8)
    *h1, dinv = _tc_layer1(x, W1, cnt)
    a1 = sc4(*h1, src2d, dst2d)
    h2 = _tc_layer_mid(a1, h1, dinv, b1.reshape(1, -1), g1.reshape(1, -1),
                       be1.reshape(1, -1), W2)
    a2 = sc4(*h2, src2d, dst2d)
    h3 = _tc_layer_mid(a2, h2, dinv, b2.reshape(1, -1), g2.reshape(1, -1),
                       be2.reshape(1, -1), W3)
    a3 = sc8(*h3, src2d, dst2d)
    pools_cnt = _tc_pool(a3, h3, dinv, b3.reshape(1, -1), batch3d, G)
    return pools_cnt[:8], pools_cnt[8]


def kernel(protein_x, protein_edge, protein_batch, mol_x, mol_edge,
           mol_batch, pW1, pb1, pg1, pbe1, pW2, pb2, pg2, pbe2, pW3, pb3,
           mW1, mb1, mg1, mbe1, mW2, mb2, mg2, mbe2, mW3, mb3,
           fW1, fb1, fW2, fb2, fW3, fb3, fW4, fb4):
    NP = protein_x.shape[0]
    NM = mol_x.shape[0]
    EP = protein_edge.shape[1]
    G = 128

    epad = _NS * _K * ((max(EP, mol_edge.shape[1]) + _NS * _K - 1)
                       // (_NS * _K))

    psrc2d, pdst2d = _pad_edges(protein_edge[0], protein_edge[1], NP, epad)
    msrc2d, mdst2d = _pad_edges(mol_edge[0], mol_edge[1], NM, epad)

    cntP, cntM = _make_sc_degree(NP, epad)(pdst2d, mdst2d)

    def padk(x, k):
        return jnp.pad(x, ((0, 0), (0, k - x.shape[1])))

    pxp = padk(protein_x, 24)
    mxp = padk(mol_x, 16)
    pW1p = jnp.pad(pW1, ((0, 2), (0, 0)))
    mW1p = jnp.pad(mW1, ((0, 1), (0, 0)))

    pbatch3d = protein_batch.reshape(NP // _BM, 1, _BM)
    mbatch3d = mol_batch.reshape(NM // _BM, 1, _BM)

    poolP, cP = _branch(pxp, psrc2d, pdst2d, cntP, pW1p, pb1, pg1, pbe1,
                        pW2, pb2, pg2, pbe2, pW3, pb3, pbatch3d,
                        NP, epad, G)
    poolM, cM = _branch(mxp, msrc2d, mdst2d, cntM, mW1p, mb1, mg1, mbe1,
                        mW2, mb2, mg2, mbe2, mW3, mb3, mbatch3d,
                        NM, epad, G)

    return _tc_head(poolP, cP, poolM, cM, fW1, fb1, fW2, fb2, fW3, fb3,
                    fW4, fb4)


# speedup vs baseline: 11.5789x; 1.7520x over previous
import functools
import math

import jax
import jax.numpy as jnp
from jax import lax
from jax.experimental import pallas as pl
from jax.experimental.pallas import tpu as pltpu
from jax.experimental.pallas import tpu_sc as plsc

_NC, _NS = 2, 16
_K = 1024
_JW = 128
_NJ = _K // _JW
_SW = 16


def _sc_mesh():
    return plsc.VectorSubcoreMesh(
        core_axis_name="c", subcore_axis_name="s",
        num_cores=_NC, num_subcores=_NS)


def _zero_rows(buf, nrows, width):
    z = jnp.zeros((16,), jnp.float32)

    def body(i, _):
        for w0 in range(0, width, 16):
            buf[i, pl.ds(w0, 16)] = z
        return 0

    lax.fori_loop(0, nrows, body, 0)


def _zero_acc(acc, zbuf, row0, nrows, zrows):
    off = 0
    while off < nrows:
        n = min(zrows, nrows - off)
        pltpu.sync_copy(zbuf.at[pl.ds(0, n)], acc.at[pl.ds(row0 + off, n)])
        off += n


def _readout(acc, out, N, R, s):
    ntiles_full = N // R
    rem = N - ntiles_full * R

    @pl.when(s < ntiles_full)
    def _():
        pltpu.sync_copy(acc.at[pl.ds(s * R, R)], out.at[pl.ds(s * R, R)])

    if rem:
        @pl.when(s == ntiles_full)
        def _():
            pltpu.sync_copy(acc.at[pl.ds(ntiles_full * R, rem)],
                            out.at[pl.ds(ntiles_full * R, rem)])


@functools.lru_cache(maxsize=None)
def _make_sc_degree(N, EPAD):
    CCH = EPAD // (_NS * _K)
    T = (CCH + 1) // 2
    R = (-(-(N + 1) // _NS) + 7) // 8 * 8
    ACC = R * _NS

    @functools.partial(
        pl.kernel,
        out_type=(jax.ShapeDtypeStruct((N, _SW), jnp.float32),
                  jax.ShapeDtypeStruct((N, _SW), jnp.float32)),
        mesh=_sc_mesh(),
        compiler_params=pltpu.CompilerParams(use_tc_tiling_on_sc=False),
        scratch_types=[
            pltpu.VMEM((_K, _SW), jnp.float32),
            pltpu.VMEM((_JW, _SW), jnp.float32),
            pltpu.VMEM((_NJ, _JW), jnp.int32),
            pltpu.VMEM((_NJ, _JW), jnp.int32),
            pltpu.VMEM_SHARED((ACC, _SW), jnp.float32),
            pltpu.SemaphoreType.DMA,
            pltpu.SemaphoreType.DMA,
        ],
    )
    def deg_kernel(dstA, dstB, outA, outB, zbuf, ones, idx0, idx1, acc,
                   sem0, sem1):
        c = lax.axis_index("c")
        s = lax.axis_index("s")

        _zero_rows(zbuf, _K, _SW)
        one = jnp.ones((16,), jnp.float32)

        def fill_ones(i, _):
            ones[i, pl.ds(0, 16)] = one
            return 0
        lax.fori_loop(0, _JW, fill_ones, 0)

        _zero_acc(acc, zbuf, s * R, R, _K)
        plsc.subcore_barrier()

        def run(dst2d, out):
            def load(idx, ch):
                rbase = (s * CCH + ch) * _NJ
                pltpu.sync_copy(dst2d.at[pl.ds(rbase, _NJ)], idx)

            def fire(idx, sem):
                for j in range(_NJ):
                    pltpu.async_copy(ones, acc.at[idx.at[j]], sem, add=True)

            def drain(idx, sem):
                for j in range(_NJ):
                    pltpu.make_async_copy(ones, acc.at[idx.at[j]], sem).wait()

            load(idx0, 0)
            fire(idx0, sem0)

            def lbody(t, _):
                c1 = 2 * t + 1
                c2 = 2 * t + 2
                c3 = 2 * t + 3

                @pl.when(c1 < CCH)
                def _():
                    load(idx1, c1)
                    fire(idx1, sem1)

                @pl.when(c2 < CCH)
                def _():
                    drain(idx0, sem0)
                    load(idx0, c2)
                    fire(idx0, sem0)

                @pl.when(c3 < CCH)
                def _():
                    drain(idx1, sem1)
                return 0

            lax.fori_loop(0, T, lbody, 0)
            if CCH >= 2:
                drain(idx1, sem1)
            drain(idx0, sem0)
            plsc.subcore_barrier()
            _readout(acc, out, N, R, s)

        @pl.when(c == 0)
        def _():
            run(dstA, outA)

        @pl.when(c == 1)
        def _():
            run(dstB, outB)

    return deg_kernel


@functools.lru_cache(maxsize=None)
def _make_sc_scatter(N, EPAD, nslabs):
    CCH = EPAD // (_NS * _K)
    T = (CCH + 1) // 2
    R = (-(-(N + 1) // _NS) + 7) // 8 * 8
    ACC = R * _NS
    out_type = tuple(jax.ShapeDtypeStruct((N, _SW), jnp.float32)
                     for _ in range(nslabs))

    @functools.partial(
        pl.kernel,
        out_type=out_type,
        mesh=_sc_mesh(),
        compiler_params=pltpu.CompilerParams(use_tc_tiling_on_sc=False),
        scratch_types=[
            pltpu.VMEM((_K, _SW), jnp.float32),
            pltpu.VMEM((_K, _SW), jnp.float32),
            pltpu.VMEM((_NJ, _JW), jnp.int32),
            pltpu.VMEM((_NJ, _JW), jnp.int32),
            pltpu.VMEM((_NJ, _JW), jnp.int32),
            pltpu.VMEM((_NJ, _JW), jnp.int32),
            pltpu.VMEM_SHARED((ACC, _SW), jnp.float32),
            pltpu.SemaphoreType.DMA,
            pltpu.SemaphoreType.DMA,
            pltpu.SemaphoreType.DMA,
            pltpu.SemaphoreType.DMA,
        ],
    )
    def scatter_kernel(*args):
        tables = args[:nslabs]
        src2d = args[nslabs]
        dst2d = args[nslabs + 1]
        outs = args[nslabs + 2:2 * nslabs + 2]
        (rows0, rows1, ixs0, ixs1, ixd0, ixd1, acc,
         gs0, gs1, ss0, ss1) = args[2 * nslabs + 2:]

        c = lax.axis_index("c")
        s = lax.axis_index("s")

        def phase(table, out):
            _zero_rows(rows0, _K, _SW)
            _zero_acc(acc, rows0, s * R, R, _K)
            plsc.subcore_barrier()

            def load(ixs, ixd, ch):
                rbase = (s * CCH + ch) * _NJ
                pltpu.sync_copy(src2d.at[pl.ds(rbase, _NJ)], ixs)
                pltpu.sync_copy(dst2d.at[pl.ds(rbase, _NJ)], ixd)

            def fire_g(ixs, rows, gsem):
                for j in range(_NJ):
                    pltpu.async_copy(table.at[ixs.at[j]],
                                     rows.at[pl.ds(j * _JW, _JW)], gsem)

            def drain_g(rows, gsem):
                pltpu.make_async_copy(out.at[pl.ds(0, _K)], rows,
                                      gsem).wait()

            def fire_s(rows, ixd, ssem):
                for j in range(_NJ):
                    pltpu.async_copy(rows.at[pl.ds(j * _JW, _JW)],
                                     acc.at[ixd.at[j]], ssem, add=True)

            def drain_s(rows, ixd, ssem):
                for j in range(_NJ):
                    pltpu.make_async_copy(rows.at[pl.ds(j * _JW, _JW)],
                                          acc.at[ixd.at[j]], ssem).wait()

            load(ixs0, ixd0, 0)
            fire_g(ixs0, rows0, gs0)

            def lbody(t, _):
                c1 = 2 * t + 1
                c2 = 2 * t + 2
                c3 = 2 * t + 3

                @pl.when(c1 < CCH)
                def _():
                    load(ixs1, ixd1, c1)
                    fire_g(ixs1, rows1, gs1)

                drain_g(rows0, gs0)
                fire_s(rows0, ixd0, ss0)

                @pl.when(c1 < CCH)
                def _():
                    drain_g(rows1, gs1)
                    fire_s(rows1, ixd1, ss1)

                @pl.when(c2 < CCH)
                def _():
                    drain_s(rows0, ixd0, ss0)
                    load(ixs0, ixd0, c2)
                    fire_g(ixs0, rows0, gs0)

                @pl.when(c3 < CCH)
                def _():
                    drain_s(rows1, ixd1, ss1)
                return 0

            lax.fori_loop(0, T, lbody, 0)
            if CCH >= 2:
                drain_s(rows1, ixd1, ss1)
            drain_s(rows0, ixd0, ss0)
            plsc.subcore_barrier()
            _readout(acc, out, N, R, s)
            plsc.subcore_barrier()

        for p in range(nslabs // _NC):
            slab_lo = p * _NC
            @pl.when(c == 0)
            def _(slab=slab_lo):
                phase(tables[slab], outs[slab])

            @pl.when(c == 1)
            def _(slab=slab_lo + 1):
                phase(tables[slab], outs[slab])

    return scatter_kernel


_BM = 1000
_BN_S = 1.0 / math.sqrt(1.0 + 1e-5)


def _tc_layer1(x, W, cnt):
    N, Kp = x.shape
    F = W.shape[1]
    nsl = F // _SW
    grid = (N // _BM,)

    def body(x_ref, w_ref, cnt_ref, *outs):
        h_refs = outs[:nsl]
        dinv_ref = outs[nsl]
        deg = cnt_ref[...][:, 0:1] + 1.0
        dinv = lax.rsqrt(deg)
        u = jnp.dot(x_ref[...], w_ref[...],
                    preferred_element_type=jnp.float32)
        hp = u * dinv
        for si in range(nsl):
            h_refs[si][...] = hp[:, si * _SW:(si + 1) * _SW]
        dinv_ref[...] = dinv

    return pl.pallas_call(
        body,
        grid=grid,
        in_specs=[
            pl.BlockSpec((_BM, Kp), lambda i: (i, 0)),
            pl.BlockSpec((Kp, F), lambda i: (0, 0)),
            pl.BlockSpec((_BM, _SW), lambda i: (i, 0)),
        ],
        out_specs=([pl.BlockSpec((_BM, _SW), lambda i: (i, 0))
                    for _ in range(nsl)]
                   + [pl.BlockSpec((_BM, 1), lambda i: (i, 0))]),
        out_shape=([jax.ShapeDtypeStruct((N, _SW), jnp.float32)
                    for _ in range(nsl)]
                  + [jax.ShapeDtypeStruct((N, 1), jnp.float32)]),
    )(x, W, cnt)


def _tc_layer_mid(aggs, hs, dinv, b, g, be, W):
    nin = len(aggs)
    N = aggs[0].shape[0]
    F = W.shape[1]
    nout = F // _SW
    grid = (N // _BM,)

    def body(*refs):
        a = refs[0:nin]
        h = refs[nin:2 * nin]
        dv = refs[2 * nin]
        b_r = refs[2 * nin + 1]
        g_r = refs[2 * nin + 2]
        be_r = refs[2 * nin + 3]
        w_r = refs[2 * nin + 4]
        outs = refs[2 * nin + 5:]
        dinv_v = dv[...]
        bias = b_r[...]
        scale = g_r[...] * _BN_S
        shift = be_r[...]
        u = jnp.zeros((_BM, F), jnp.float32)
        for si in range(nin):
            lo, hi = si * _SW, (si + 1) * _SW
            z = dinv_v * (a[si][...] + h[si][...]) + bias[:, lo:hi]
            x = jnp.maximum(z * scale[:, lo:hi] + shift[:, lo:hi], 0.0)
            u = u + jnp.dot(x, w_r[...][lo:hi, :],
                            preferred_element_type=jnp.float32)
        hp = u * dinv_v
        for si in range(nout):
            outs[si][...] = hp[:, si * _SW:(si + 1) * _SW]

    return pl.pallas_call(
        body,
        grid=grid,
        in_specs=(
            [pl.BlockSpec((_BM, _SW), lambda i: (i, 0))
             for _ in range(2 * nin)]
            + [pl.BlockSpec((_BM, 1), lambda i: (i, 0)),
               pl.BlockSpec((1, nin * _SW), lambda i: (0, 0)),
               pl.BlockSpec((1, nin * _SW), lambda i: (0, 0)),
               pl.BlockSpec((1, nin * _SW), lambda i: (0, 0)),
               pl.BlockSpec((nin * _SW, F), lambda i: (0, 0))]
        ),
        out_specs=[pl.BlockSpec((_BM, _SW), lambda i: (i, 0))
                   for _ in range(nout)],
        out_shape=[jax.ShapeDtypeStruct((N, _SW), jnp.float32)
                   for _ in range(nout)],
    )(*aggs, *hs, dinv, b, g, be, W)


def _tc_pool(aggs, hs, dinv, b3, batch3d, G):
    N = aggs[0].shape[0]
    grid = (N // _BM,)
    nsl = len(aggs)

    def body(*refs):
        a = refs[0:nsl]
        h = refs[nsl:2 * nsl]
        dv = refs[2 * nsl]
        b_r = refs[2 * nsl + 1]
        bt = refs[2 * nsl + 2]
        pouts = refs[2 * nsl + 3:2 * nsl + 3 + nsl]
        cnt_o = refs[2 * nsl + 3 + nsl]

        @pl.when(pl.program_id(0) == 0)
        def _():
            for po in pouts:
                po[...] = jnp.zeros_like(po)
            cnt_o[...] = jnp.zeros_like(cnt_o)

        dinv_v = dv[...]
        bias = b_r[...]
        seg = bt[...].reshape(1, _BM)
        gid = lax.broadcasted_iota(jnp.int32, (G, _BM), 0)
        mask = (seg == gid).astype(jnp.float32)
        cnt_o[...] += jnp.sum(mask, axis=1, keepdims=True)
        for si in range(nsl):
            lo, hi = si * _SW, (si + 1) * _SW
            x4 = jnp.maximum(
                dinv_v * (a[si][...] + h[si][...]) + bias[:, lo:hi], 0.0)
            pouts[si][...] += jnp.dot(mask, x4,
                                      preferred_element_type=jnp.float32)

    return pl.pallas_call(
        body,
        grid=grid,
        in_specs=(
            [pl.BlockSpec((_BM, _SW), lambda i: (i, 0))
             for _ in range(2 * nsl)]
            + [pl.BlockSpec((_BM, 1), lambda i: (i, 0)),
               pl.BlockSpec((1, nsl * _SW), lambda i: (0, 0)),
               pl.BlockSpec((1, 1, _BM), lambda i: (i, 0, 0))]
        ),
        out_specs=([pl.BlockSpec((G, _SW), lambda i: (0, 0))
                    for _ in range(nsl)]
                   + [pl.BlockSpec((G, 1), lambda i: (0, 0))]),
        out_shape=([jax.ShapeDtypeStruct((G, _SW), jnp.float32)
                    for _ in range(nsl)]
                  + [jax.ShapeDtypeStruct((G, 1), jnp.float32)]),
    )(*aggs, *hs, dinv, b3, batch3d)


def _tc_head(poolP, cntP, poolM, cntM, fW1, fb1, fW2, fb2, fW3, fb3,
             fW4, fb4):
    G = cntP.shape[0]
    nsl = len(poolP)

    def body(*refs):
        pP = refs[0:nsl]
        cP = refs[nsl]
        pM = refs[nsl + 1:2 * nsl + 1]
        cM = refs[2 * nsl + 1]
        w1, b1, w2, b2, w3, b3, w4, b4, out = refs[2 * nsl + 2:]
        invP = 1.0 / jnp.maximum(cP[...], 1.0)
        invM = 1.0 / jnp.maximum(cM[...], 1.0)
        acc = jnp.zeros((G, 128), jnp.float32)
        half = nsl * _SW
        for si in range(nsl):
            acc += jnp.dot(pP[si][...] * invP,
                           w1[...][si * _SW:(si + 1) * _SW, :],
                           preferred_element_type=jnp.float32)
            acc += jnp.dot(pM[si][...] * invM,
                           w1[...][half + si * _SW:half + (si + 1) * _SW, :],
                           preferred_element_type=jnp.float32)
        h = jnp.maximum(acc + b1[...], 0.0)
        h = jnp.maximum(jnp.dot(h, w2[...], preferred_element_type=jnp.float32)
                        + b2[...], 0.0)
        h = jnp.maximum(jnp.dot(h, w3[...], preferred_element_type=jnp.float32)
                        + b3[...], 0.0)
        out[...] = (jnp.dot(h, w4[...], preferred_element_type=jnp.float32)
                    + b4[...])

    full = lambda shape: pl.BlockSpec(shape, lambda: tuple(0 for _ in shape))
    ins = ([full((G, _SW)) for _ in range(nsl)] + [full((G, 1))]
           + [full((G, _SW)) for _ in range(nsl)] + [full((G, 1))]
           + [full((256, 128)), full((1, 128)), full((128, 64)),
              full((1, 64)), full((64, 32)), full((1, 32)),
              full((32, 1)), full((1, 1))])
    return pl.pallas_call(
        body,
        grid=(),
        in_specs=ins,
        out_specs=full((G, 1)),
        out_shape=jax.ShapeDtypeStruct((G, 1), jnp.float32),
    )(*poolP, cntP, *poolM, cntM, fW1, fb1.reshape(1, -1), fW2,
      fb2.reshape(1, -1), fW3, fb3.reshape(1, -1), fW4, fb4.reshape(1, -1))


def _pad_edges(src, dst, N, EPAD):
    E = src.shape[0]
    pad = EPAD - E
    srcp = jnp.concatenate([src, jnp.zeros((pad,), jnp.int32)])
    dstp = jnp.concatenate([dst, jnp.full((pad,), N, jnp.int32)])
    return srcp.reshape(EPAD // _JW, _JW), dstp.reshape(EPAD // _JW, _JW)


def _branch(x, src2d, dst2d, cnt, W1, b1, g1, be1, W2, b2, g2, be2, W3, b3,
            batch3d, N, EPAD, G):
    sc4 = _make_sc_scatter(N, EPAD, 4)
    sc8 = _make_sc_scatter(N, EPAD, 8)
    *h1, dinv = _tc_layer1(x, W1, cnt)
    a1 = sc4(*h1, src2d, dst2d)
    h2 = _tc_layer_mid(a1, h1, dinv, b1.reshape(1, -1), g1.reshape(1, -1),
                       be1.reshape(1, -1), W2)
    a2 = sc4(*h2, src2d, dst2d)
    h3 = _tc_layer_mid(a2, h2, dinv, b2.reshape(1, -1), g2.reshape(1, -1),
                       be2.reshape(1, -1), W3)
    a3 = sc8(*h3, src2d, dst2d)
    pools_cnt = _tc_pool(a3, h3, dinv, b3.reshape(1, -1), batch3d, G)
    return pools_cnt[:8], pools_cnt[8]


def kernel(protein_x, protein_edge, protein_batch, mol_x, mol_edge,
           mol_batch, pW1, pb1, pg1, pbe1, pW2, pb2, pg2, pbe2, pW3, pb3,
           mW1, mb1, mg1, mbe1, mW2, mb2, mg2, mbe2, mW3, mb3,
           fW1, fb1, fW2, fb2, fW3, fb3, fW4, fb4):
    NP = protein_x.shape[0]
    NM = mol_x.shape[0]
    EP = protein_edge.shape[1]
    G = 128

    epad = _NS * _K * ((max(EP, mol_edge.shape[1]) + _NS * _K - 1)
                       // (_NS * _K))

    psrc2d, pdst2d = _pad_edges(protein_edge[0], protein_edge[1], NP, epad)
    msrc2d, mdst2d = _pad_edges(mol_edge[0], mol_edge[1], NM, epad)

    cntP, cntM = _make_sc_degree(NP, epad)(pdst2d, mdst2d)

    def padk(x, k):
        return jnp.pad(x, ((0, 0), (0, k - x.shape[1])))

    pxp = padk(protein_x, 24)
    mxp = padk(mol_x, 16)
    pW1p = jnp.pad(pW1, ((0, 2), (0, 0)))
    mW1p = jnp.pad(mW1, ((0, 1), (0, 0)))

    pbatch3d = protein_batch.reshape(NP // _BM, 1, _BM)
    mbatch3d = mol_batch.reshape(NM // _BM, 1, _BM)

    poolP, cP = _branch(pxp, psrc2d, pdst2d, cntP, pW1p, pb1, pg1, pbe1,
                        pW2, pb2, pg2, pbe2, pW3, pb3, pbatch3d,
                        NP, epad, G)
    poolM, cM = _branch(mxp, msrc2d, mdst2d, cntM, mW1p, mb1, mg1, mbe1,
                        mW2, mb2, mg2, mbe2, mW3, mb3, mbatch3d,
                        NM, epad, G)

    return _tc_head(poolP, cP, poolM, cM, fW1, fb1, fW2, fb2, fW3, fb3,
                    fW4, fb4)


# speedup vs baseline: 12.2426x; 1.0573x over previous
import functools
import math

import jax
import jax.numpy as jnp
from jax import lax
from jax.experimental import pallas as pl
from jax.experimental.pallas import tpu as pltpu
from jax.experimental.pallas import tpu_sc as plsc

_NC, _NS = 2, 16
_K = 1024
_JW = 128
_NJ = _K // _JW
_SW = 16


def _sc_mesh():
    return plsc.VectorSubcoreMesh(
        core_axis_name="c", subcore_axis_name="s",
        num_cores=_NC, num_subcores=_NS)


def _zero_rows(buf, nrows, width):
    z = jnp.zeros((16,), jnp.float32)

    def body(i, _):
        for w0 in range(0, width, 16):
            buf[i, pl.ds(w0, 16)] = z
        return 0

    lax.fori_loop(0, nrows, body, 0)


def _zero_acc(acc, zbuf, row0, nrows, zrows):
    off = 0
    while off < nrows:
        n = min(zrows, nrows - off)
        pltpu.sync_copy(zbuf.at[pl.ds(0, n)], acc.at[pl.ds(row0 + off, n)])
        off += n


def _readout(acc, out, N, R, s):
    ntiles_full = N // R
    rem = N - ntiles_full * R

    @pl.when(s < ntiles_full)
    def _():
        pltpu.sync_copy(acc.at[pl.ds(s * R, R)], out.at[pl.ds(s * R, R)])

    if rem:
        @pl.when(s == ntiles_full)
        def _():
            pltpu.sync_copy(acc.at[pl.ds(ntiles_full * R, rem)],
                            out.at[pl.ds(ntiles_full * R, rem)])


@functools.lru_cache(maxsize=None)
def _make_sc_degree(N, EPAD):
    CCH = EPAD // (_NS * _K)
    T = (CCH + 1) // 2
    R = (-(-(N + 1) // _NS) + 7) // 8 * 8
    ACC = R * _NS

    @functools.partial(
        pl.kernel,
        out_type=(jax.ShapeDtypeStruct((N, _SW), jnp.float32),
                  jax.ShapeDtypeStruct((N, _SW), jnp.float32)),
        mesh=_sc_mesh(),
        compiler_params=pltpu.CompilerParams(use_tc_tiling_on_sc=False),
        scratch_types=[
            pltpu.VMEM((_K, _SW), jnp.float32),
            pltpu.VMEM((_JW, _SW), jnp.float32),
            pltpu.VMEM((_NJ, _JW), jnp.int32),
            pltpu.VMEM((_NJ, _JW), jnp.int32),
            pltpu.VMEM_SHARED((ACC, _SW), jnp.float32),
            pltpu.SemaphoreType.DMA,
            pltpu.SemaphoreType.DMA,
        ],
    )
    def deg_kernel(dstA, dstB, outA, outB, zbuf, ones, idx0, idx1, acc,
                   sem0, sem1):
        c = lax.axis_index("c")
        s = lax.axis_index("s")

        _zero_rows(zbuf, _K, _SW)
        one = jnp.ones((16,), jnp.float32)

        def fill_ones(i, _):
            ones[i, pl.ds(0, 16)] = one
            return 0
        lax.fori_loop(0, _JW, fill_ones, 0)

        _zero_acc(acc, zbuf, s * R, R, _K)
        plsc.subcore_barrier()

        def run(dst2d, out):
            def load(idx, ch):
                rbase = (s * CCH + ch) * _NJ
                pltpu.sync_copy(dst2d.at[pl.ds(rbase, _NJ)], idx)

            def fire(idx, sem):
                for j in range(_NJ):
                    pltpu.async_copy(ones, acc.at[idx.at[j]], sem, add=True)

            def drain(idx, sem):
                for j in range(_NJ):
                    pltpu.make_async_copy(ones, acc.at[idx.at[j]], sem).wait()

            load(idx0, 0)
            fire(idx0, sem0)

            def lbody(t, _):
                c1 = 2 * t + 1
                c2 = 2 * t + 2
                c3 = 2 * t + 3

                @pl.when(c1 < CCH)
                def _():
                    load(idx1, c1)
                    fire(idx1, sem1)

                @pl.when(c2 < CCH)
                def _():
                    drain(idx0, sem0)
                    load(idx0, c2)
                    fire(idx0, sem0)

                @pl.when(c3 < CCH)
                def _():
                    drain(idx1, sem1)
                return 0

            lax.fori_loop(0, T, lbody, 0)
            if CCH >= 2:
                drain(idx1, sem1)
            drain(idx0, sem0)
            plsc.subcore_barrier()
            _readout(acc, out, N, R, s)

        @pl.when(c == 0)
        def _():
            run(dstA, outA)

        @pl.when(c == 1)
        def _():
            run(dstB, outB)

    return deg_kernel


@functools.lru_cache(maxsize=None)
def _make_sc_scatter(N, EPAD, nslabs):
    CCH = EPAD // (_NS * _K)
    T = (CCH + 1) // 2
    R = (-(-(N + 1) // _NS) + 7) // 8 * 8
    ACC = R * _NS
    out_type = tuple(jax.ShapeDtypeStruct((N, _SW), jnp.float32)
                     for _ in range(nslabs))

    @functools.partial(
        pl.kernel,
        out_type=out_type,
        mesh=_sc_mesh(),
        compiler_params=pltpu.CompilerParams(use_tc_tiling_on_sc=False),
        scratch_types=[
            pltpu.VMEM((_K, _SW), jnp.float32),
            pltpu.VMEM((_K, _SW), jnp.float32),
            pltpu.VMEM((_NJ, 2, _JW), jnp.int32),
            pltpu.VMEM((_NJ, 2, _JW), jnp.int32),
            pltpu.VMEM_SHARED((ACC, _SW), jnp.float32),
            pltpu.SemaphoreType.DMA,
            pltpu.SemaphoreType.DMA,
            pltpu.SemaphoreType.DMA,
            pltpu.SemaphoreType.DMA,
            pltpu.SemaphoreType.DMA,
            pltpu.SemaphoreType.DMA,
        ],
    )
    def scatter_kernel(*args):
        tables = args[:nslabs]
        edges = args[nslabs]
        outs = args[nslabs + 1:2 * nslabs + 1]
        (rows0, rows1, ix0, ix1, acc,
         is0, is1, gs0, gs1, ss0, ss1) = args[2 * nslabs + 1:]

        c = lax.axis_index("c")
        s = lax.axis_index("s")

        def phase(table, out):
            _zero_rows(rows0, _K, _SW)
            _zero_acc(acc, rows0, s * R, R, _K)
            plsc.subcore_barrier()

            def fire_l(ix, isem, ch):
                rbase = (s * CCH + ch) * _NJ
                pltpu.async_copy(edges.at[pl.ds(rbase, _NJ)], ix, isem)

            def drain_l(ix, isem):
                pltpu.make_async_copy(edges.at[pl.ds(0, _NJ)], ix,
                                      isem).wait()

            def fire_g(ix, rows, gsem):
                for j in range(_NJ):
                    pltpu.async_copy(table.at[ix.at[j, 0]],
                                     rows.at[pl.ds(j * _JW, _JW)], gsem)

            def drain_g(rows, gsem):
                pltpu.make_async_copy(out.at[pl.ds(0, _K)], rows,
                                      gsem).wait()

            def fire_s(rows, ix, ssem):
                for j in range(_NJ):
                    pltpu.async_copy(rows.at[pl.ds(j * _JW, _JW)],
                                     acc.at[ix.at[j, 1]], ssem, add=True)

            def drain_s(rows, ix, ssem):
                for j in range(_NJ):
                    pltpu.make_async_copy(rows.at[pl.ds(j * _JW, _JW)],
                                          acc.at[ix.at[j, 1]], ssem).wait()

            fire_l(ix0, is0, 0)
            drain_l(ix0, is0)
            fire_g(ix0, rows0, gs0)

            def lbody(t, _):
                c1 = 2 * t + 1
                c2 = 2 * t + 2
                c3 = 2 * t + 3

                @pl.when(c1 < CCH)
                def _():
                    fire_l(ix1, is1, c1)

                drain_g(rows0, gs0)
                fire_s(rows0, ix0, ss0)

                @pl.when(c1 < CCH)
                def _():
                    drain_l(ix1, is1)
                    fire_g(ix1, rows1, gs1)

                @pl.when(c2 < CCH)
                def _():
                    drain_s(rows0, ix0, ss0)
                    fire_l(ix0, is0, c2)

                @pl.when(c1 < CCH)
                def _():
                    drain_g(rows1, gs1)
                    fire_s(rows1, ix1, ss1)

                @pl.when(c2 < CCH)
                def _():
                    drain_l(ix0, is0)
                    fire_g(ix0, rows0, gs0)

                @pl.when(c3 < CCH)
                def _():
                    drain_s(rows1, ix1, ss1)
                return 0

            lax.fori_loop(0, T, lbody, 0)
            if CCH >= 2:
                drain_s(rows1, ix1, ss1)
            drain_s(rows0, ix0, ss0)
            plsc.subcore_barrier()
            _readout(acc, out, N, R, s)
            plsc.subcore_barrier()

        for p in range(nslabs // _NC):
            slab_lo = p * _NC
            @pl.when(c == 0)
            def _(slab=slab_lo):
                phase(tables[slab], outs[slab])

            @pl.when(c == 1)
            def _(slab=slab_lo + 1):
                phase(tables[slab], outs[slab])

    return scatter_kernel


_BM = 1000
_BN_S = 1.0 / math.sqrt(1.0 + 1e-5)


def _tc_layer1(x, W, cnt):
    N, Kp = x.shape
    F = W.shape[1]
    nsl = F // _SW
    grid = (N // _BM,)

    def body(x_ref, w_ref, cnt_ref, *outs):
        h_refs = outs[:nsl]
        dinv_ref = outs[nsl]
        deg = cnt_ref[...][:, 0:1] + 1.0
        dinv = lax.rsqrt(deg)
        u = jnp.dot(x_ref[...], w_ref[...],
                    preferred_element_type=jnp.float32)
        hp = u * dinv
        for si in range(nsl):
            h_refs[si][...] = hp[:, si * _SW:(si + 1) * _SW]
        dinv_ref[...] = dinv

    return pl.pallas_call(
        body,
        grid=grid,
        in_specs=[
            pl.BlockSpec((_BM, Kp), lambda i: (i, 0)),
            pl.BlockSpec((Kp, F), lambda i: (0, 0)),
            pl.BlockSpec((_BM, _SW), lambda i: (i, 0)),
        ],
        out_specs=([pl.BlockSpec((_BM, _SW), lambda i: (i, 0))
                    for _ in range(nsl)]
                   + [pl.BlockSpec((_BM, 1), lambda i: (i, 0))]),
        out_shape=([jax.ShapeDtypeStruct((N, _SW), jnp.float32)
                    for _ in range(nsl)]
                  + [jax.ShapeDtypeStruct((N, 1), jnp.float32)]),
    )(x, W, cnt)


def _tc_layer_mid(aggs, hs, dinv, b, g, be, W):
    nin = len(aggs)
    N = aggs[0].shape[0]
    F = W.shape[1]
    nout = F // _SW
    grid = (N // _BM,)

    def body(*refs):
        a = refs[0:nin]
        h = refs[nin:2 * nin]
        dv = refs[2 * nin]
        b_r = refs[2 * nin + 1]
        g_r = refs[2 * nin + 2]
        be_r = refs[2 * nin + 3]
        w_r = refs[2 * nin + 4]
        outs = refs[2 * nin + 5:]
        dinv_v = dv[...]
        bias = b_r[...]
        scale = g_r[...] * _BN_S
        shift = be_r[...]
        u = jnp.zeros((_BM, F), jnp.float32)
        for si in range(nin):
            lo, hi = si * _SW, (si + 1) * _SW
            z = dinv_v * (a[si][...] + h[si][...]) + bias[:, lo:hi]
            x = jnp.maximum(z * scale[:, lo:hi] + shift[:, lo:hi], 0.0)
            u = u + jnp.dot(x, w_r[...][lo:hi, :],
                            preferred_element_type=jnp.float32)
        hp = u * dinv_v
        for si in range(nout):
            outs[si][...] = hp[:, si * _SW:(si + 1) * _SW]

    return pl.pallas_call(
        body,
        grid=grid,
        in_specs=(
            [pl.BlockSpec((_BM, _SW), lambda i: (i, 0))
             for _ in range(2 * nin)]
            + [pl.BlockSpec((_BM, 1), lambda i: (i, 0)),
               pl.BlockSpec((1, nin * _SW), lambda i: (0, 0)),
               pl.BlockSpec((1, nin * _SW), lambda i: (0, 0)),
               pl.BlockSpec((1, nin * _SW), lambda i: (0, 0)),
               pl.BlockSpec((nin * _SW, F), lambda i: (0, 0))]
        ),
        out_specs=[pl.BlockSpec((_BM, _SW), lambda i: (i, 0))
                   for _ in range(nout)],
        out_shape=[jax.ShapeDtypeStruct((N, _SW), jnp.float32)
                   for _ in range(nout)],
    )(*aggs, *hs, dinv, b, g, be, W)


def _tc_pool(aggs, hs, dinv, b3, batch3d, G):
    N = aggs[0].shape[0]
    grid = (N // _BM,)
    nsl = len(aggs)

    def body(*refs):
        a = refs[0:nsl]
        h = refs[nsl:2 * nsl]
        dv = refs[2 * nsl]
        b_r = refs[2 * nsl + 1]
        bt = refs[2 * nsl + 2]
        pouts = refs[2 * nsl + 3:2 * nsl + 3 + nsl]
        cnt_o = refs[2 * nsl + 3 + nsl]

        @pl.when(pl.program_id(0) == 0)
        def _():
            for po in pouts:
                po[...] = jnp.zeros_like(po)
            cnt_o[...] = jnp.zeros_like(cnt_o)

        dinv_v = dv[...]
        bias = b_r[...]
        seg = bt[...].reshape(1, _BM)
        gid = lax.broadcasted_iota(jnp.int32, (G, _BM), 0)
        mask = (seg == gid).astype(jnp.float32)
        cnt_o[...] += jnp.sum(mask, axis=1, keepdims=True)
        for si in range(nsl):
            lo, hi = si * _SW, (si + 1) * _SW
            x4 = jnp.maximum(
                dinv_v * (a[si][...] + h[si][...]) + bias[:, lo:hi], 0.0)
            pouts[si][...] += jnp.dot(mask, x4,
                                      preferred_element_type=jnp.float32)

    return pl.pallas_call(
        body,
        grid=grid,
        in_specs=(
            [pl.BlockSpec((_BM, _SW), lambda i: (i, 0))
             for _ in range(2 * nsl)]
            + [pl.BlockSpec((_BM, 1), lambda i: (i, 0)),
               pl.BlockSpec((1, nsl * _SW), lambda i: (0, 0)),
               pl.BlockSpec((1, 1, _BM), lambda i: (i, 0, 0))]
        ),
        out_specs=([pl.BlockSpec((G, _SW), lambda i: (0, 0))
                    for _ in range(nsl)]
                   + [pl.BlockSpec((G, 1), lambda i: (0, 0))]),
        out_shape=([jax.ShapeDtypeStruct((G, _SW), jnp.float32)
                    for _ in range(nsl)]
                  + [jax.ShapeDtypeStruct((G, 1), jnp.float32)]),
    )(*aggs, *hs, dinv, b3, batch3d)


def _tc_head(poolP, cntP, poolM, cntM, fW1, fb1, fW2, fb2, fW3, fb3,
             fW4, fb4):
    G = cntP.shape[0]
    nsl = len(poolP)

    def body(*refs):
        pP = refs[0:nsl]
        cP = refs[nsl]
        pM = refs[nsl + 1:2 * nsl + 1]
        cM = refs[2 * nsl + 1]
        w1, b1, w2, b2, w3, b3, w4, b4, out = refs[2 * nsl + 2:]
        invP = 1.0 / jnp.maximum(cP[...], 1.0)
        invM = 1.0 / jnp.maximum(cM[...], 1.0)
        acc = jnp.zeros((G, 128), jnp.float32)
        half = nsl * _SW
        for si in range(nsl):
            acc += jnp.dot(pP[si][...] * invP,
                           w1[...][si * _SW:(si + 1) * _SW, :],
                           preferred_element_type=jnp.float32)
            acc += jnp.dot(pM[si][...] * invM,
                           w1[...][half + si * _SW:half + (si + 1) * _SW, :],
                           preferred_element_type=jnp.float32)
        h = jnp.maximum(acc + b1[...], 0.0)
        h = jnp.maximum(jnp.dot(h, w2[...], preferred_element_type=jnp.float32)
                        + b2[...], 0.0)
        h = jnp.maximum(jnp.dot(h, w3[...], preferred_element_type=jnp.float32)
                        + b3[...], 0.0)
        out[...] = (jnp.dot(h, w4[...], preferred_element_type=jnp.float32)
                    + b4[...])

    full = lambda shape: pl.BlockSpec(shape, lambda: tuple(0 for _ in shape))
    ins = ([full((G, _SW)) for _ in range(nsl)] + [full((G, 1))]
           + [full((G, _SW)) for _ in range(nsl)] + [full((G, 1))]
           + [full((256, 128)), full((1, 128)), full((128, 64)),
              full((1, 64)), full((64, 32)), full((1, 32)),
              full((32, 1)), full((1, 1))])
    return pl.pallas_call(
        body,
        grid=(),
        in_specs=ins,
        out_specs=full((G, 1)),
        out_shape=jax.ShapeDtypeStruct((G, 1), jnp.float32),
    )(*poolP, cntP, *poolM, cntM, fW1, fb1.reshape(1, -1), fW2,
      fb2.reshape(1, -1), fW3, fb3.reshape(1, -1), fW4, fb4.reshape(1, -1))


def _pad_edges(src, dst, N, EPAD):
    E = src.shape[0]
    pad = EPAD - E
    srcp = jnp.concatenate([src, jnp.zeros((pad,), jnp.int32)])
    dstp = jnp.concatenate([dst, jnp.full((pad,), N, jnp.int32)])
    return srcp.reshape(EPAD // _JW, _JW), dstp.reshape(EPAD // _JW, _JW)


def _branch(x, src2d, dst2d, cnt, W1, b1, g1, be1, W2, b2, g2, be2, W3, b3,
            batch3d, N, EPAD, G):
    sc4 = _make_sc_scatter(N, EPAD, 4)
    sc8 = _make_sc_scatter(N, EPAD, 8)
    edges = jnp.stack([src2d, dst2d], axis=1)
    *h1, dinv = _tc_layer1(x, W1, cnt)
    a1 = sc4(*h1, edges)
    h2 = _tc_layer_mid(a1, h1, dinv, b1.reshape(1, -1), g1.reshape(1, -1),
                       be1.reshape(1, -1), W2)
    a2 = sc4(*h2, edges)
    h3 = _tc_layer_mid(a2, h2, dinv, b2.reshape(1, -1), g2.reshape(1, -1),
                       be2.reshape(1, -1), W3)
    a3 = sc8(*h3, edges)
    pools_cnt = _tc_pool(a3, h3, dinv, b3.reshape(1, -1), batch3d, G)
    return pools_cnt[:8], pools_cnt[8]


def kernel(protein_x, protein_edge, protein_batch, mol_x, mol_edge,
           mol_batch, pW1, pb1, pg1, pbe1, pW2, pb2, pg2, pbe2, pW3, pb3,
           mW1, mb1, mg1, mbe1, mW2, mb2, mg2, mbe2, mW3, mb3,
           fW1, fb1, fW2, fb2, fW3, fb3, fW4, fb4):
    NP = protein_x.shape[0]
    NM = mol_x.shape[0]
    EP = protein_edge.shape[1]
    G = 128

    epad = _NS * _K * ((max(EP, mol_edge.shape[1]) + _NS * _K - 1)
                       // (_NS * _K))

    psrc2d, pdst2d = _pad_edges(protein_edge[0], protein_edge[1], NP, epad)
    msrc2d, mdst2d = _pad_edges(mol_edge[0], mol_edge[1], NM, epad)

    cntP, cntM = _make_sc_degree(NP, epad)(pdst2d, mdst2d)

    def padk(x, k):
        return jnp.pad(x, ((0, 0), (0, k - x.shape[1])))

    pxp = padk(protein_x, 24)
    mxp = padk(mol_x, 16)
    pW1p = jnp.pad(pW1, ((0, 2), (0, 0)))
    mW1p = jnp.pad(mW1, ((0, 1), (0, 0)))

    pbatch3d = protein_batch.reshape(NP // _BM, 1, _BM)
    mbatch3d = mol_batch.reshape(NM // _BM, 1, _BM)

    poolP, cP = _branch(pxp, psrc2d, pdst2d, cntP, pW1p, pb1, pg1, pbe1,
                        pW2, pb2, pg2, pbe2, pW3, pb3, pbatch3d,
                        NP, epad, G)
    poolM, cM = _branch(mxp, msrc2d, mdst2d, cntM, mW1p, mb1, mg1, mbe1,
                        mW2, mb2, mg2, mbe2, mW3, mb3, mbatch3d,
                        NM, epad, G)

    return _tc_head(poolP, cP, poolM, cM, fW1, fb1, fW2, fb2, fW3, fb3,
                    fW4, fb4)
